# baseline (speedup 1.0000x reference)
import math

import jax
import jax.numpy as jnp
from jax import lax
from jax.experimental import pallas as pl
from jax.experimental.pallas import tpu as pltpu

ROWS = 128
COLS = 100000
TEMP = 5.0
BC = 2048
NCB = math.ceil(COLS / BC)

_KS0 = 0
_KS1 = 42
_KS2 = 42 ^ 0x1BD11BDA
_ROT_A = (13, 15, 26, 6)
_ROT_B = (17, 29, 16, 24)


def _rounds(x0, x1, rots):
    for r in rots:
        x0 = x0 + x1
        x1 = (x1 << r) | lax.shift_right_logical(x1, 32 - r)
        x1 = x1 ^ x0
    return x0, x1


def _threefry_bits(e):
    x0 = jnp.zeros_like(e) + _KS0
    x1 = e + _KS1
    x0, x1 = _rounds(x0, x1, _ROT_A)
    x0, x1 = x0 + _KS1, x1 + (_KS2 + 1)
    x0, x1 = _rounds(x0, x1, _ROT_B)
    x0, x1 = x0 + _KS2, x1 + (_KS0 + 2)
    x0, x1 = _rounds(x0, x1, _ROT_A)
    x0, x1 = x0 + _KS0, x1 + (_KS1 + 3)
    x0, x1 = _rounds(x0, x1, _ROT_B)
    x0, x1 = x0 + _KS1, x1 + (_KS2 + 4)
    x0, x1 = _rounds(x0, x1, _ROT_A)
    x0, x1 = x0 + _KS2, x1 + (_KS0 + 5)
    return x0 ^ x1


def _perturbed(x_block, cb):
    jj = lax.broadcasted_iota(jnp.int32, (ROWS, BC), 1) + cb * BC
    ii = lax.broadcasted_iota(jnp.int32, (ROWS, BC), 0)
    e = ii * COLS + jj
    bits = _threefry_bits(e)
    mant = lax.shift_right_logical(bits, 9) | 0x3F800000
    u = lax.bitcast_convert_type(mant, jnp.float32) - 1.0
    g = -jnp.log(-jnp.log(u + 1e-8) + 1e-8)
    w = (x_block + g) / TEMP
    w = jnp.where(jj < COLS, w, -jnp.inf)
    return w, jj


def _argmax_kernel(x_ref, idx_ref, val_ref):
    cb = pl.program_id(0)
    w, jj = _perturbed(x_ref[...], cb)
    m = jnp.max(w, axis=1, keepdims=True)
    idxb = jnp.min(
        jnp.where(w == m, jj, jnp.int32(2**31 - 1)), axis=1, keepdims=True
    )

    @pl.when(cb == 0)
    def _():
        val_ref[...] = m
        idx_ref[...] = idxb

    @pl.when(cb > 0)
    def _():
        better = m > val_ref[...]
        val_ref[...] = jnp.where(better, m, val_ref[...])
        idx_ref[...] = jnp.where(better, idxb, idx_ref[...])


def _onehot_kernel(idx_ref, out_ref):
    cb = pl.program_id(0)
    jj = lax.broadcasted_iota(jnp.int32, (ROWS, BC), 1) + cb * BC
    out_ref[...] = jnp.where(jj == idx_ref[...], 1.0, 0.0).astype(jnp.float32)


@jax.jit
def kernel(logits):
    idx = pl.pallas_call(
        _argmax_kernel,
        grid=(NCB,),
        in_specs=[pl.BlockSpec((ROWS, BC), lambda cb: (0, cb))],
        out_specs=pl.BlockSpec((ROWS, 1), lambda cb: (0, 0)),
        out_shape=jax.ShapeDtypeStruct((ROWS, 1), jnp.int32),
        scratch_shapes=[pltpu.VMEM((ROWS, 1), jnp.float32)],
    )(logits)
    out = pl.pallas_call(
        _onehot_kernel,
        grid=(NCB,),
        in_specs=[pl.BlockSpec((ROWS, 1), lambda cb: (0, 0))],
        out_specs=pl.BlockSpec((ROWS, BC), lambda cb: (0, cb)),
        out_shape=jax.ShapeDtypeStruct((ROWS, COLS), jnp.float32),
    )(idx)
    return out
